# baseline (speedup 1.0000x reference)
import functools

import jax
import jax.numpy as jnp
from jax import lax
from jax.experimental import pallas as pl
from jax.experimental.pallas import tpu as pltpu
from jax.experimental.pallas import tpu_sc as plsc

N = 10000
DEG = 32
E = N * DEG
HF = 128
NH = 8
NF = 16
NEG_SLOPE = 0.2

ROWS_TC = 2000

NC, NS = 2, 16
NW = NC * NS
B = 4
EB = B * DEG
NB = N // B
BASE_CNT = NB // NW
EXTRA = NB % NW
PW = BASE_CNT + 1


def _tc_body(feat_ref, wp_ref, alp_ref, arp_ref, h_ref, ar_ref, ac2_ref):
    h = jnp.dot(feat_ref[...], wp_ref[...],
                preferred_element_type=jnp.float32,
                precision=lax.Precision.HIGHEST)
    h_ref[...] = h.astype(jnp.bfloat16)
    ar_ref[...] = jnp.dot(h, alp_ref[...],
                          preferred_element_type=jnp.float32,
                          precision=lax.Precision.HIGHEST)
    acv = jnp.dot(h, arp_ref[...],
                  preferred_element_type=jnp.float32,
                  precision=lax.Precision.HIGHEST)
    ac2_ref[...] = jnp.concatenate([acv, acv], axis=1)


_tc_call = pl.pallas_call(
    _tc_body,
    grid=(N // ROWS_TC,),
    in_specs=[
        pl.BlockSpec((ROWS_TC, HF), lambda i: (i, 0)),
        pl.BlockSpec((HF, HF), lambda i: (0, 0)),
        pl.BlockSpec((HF, NH), lambda i: (0, 0)),
        pl.BlockSpec((HF, NH), lambda i: (0, 0)),
    ],
    out_specs=[
        pl.BlockSpec((ROWS_TC, HF), lambda i: (i, 0)),
        pl.BlockSpec((ROWS_TC, NH), lambda i: (i, 0)),
        pl.BlockSpec((ROWS_TC, 2 * NH), lambda i: (i, 0)),
    ],
    out_shape=[
        jax.ShapeDtypeStruct((N, HF), jnp.bfloat16),
        jax.ShapeDtypeStruct((N, NH), jnp.float32),
        jax.ShapeDtypeStruct((N, 2 * NH), jnp.float32),
    ],
)


def _tree_reduce(op, xs):
    xs = list(xs)
    while len(xs) > 1:
        nxt = [op(xs[i], xs[i + 1]) for i in range(0, len(xs) - 1, 2)]
        if len(xs) % 2:
            nxt.append(xs[-1])
        xs = nxt
    return xs[0]


@functools.partial(
    pl.kernel,
    mesh=plsc.VectorSubcoreMesh(core_axis_name="c", subcore_axis_name="s"),
    out_type=jax.ShapeDtypeStruct((N * HF,), jnp.float32),
    compiler_params=pltpu.CompilerParams(needs_layout_passes=False,
                                         use_tc_tiling_on_sc=False),
    scratch_types=[
        pltpu.VMEM((PW * EB,), jnp.int32),
        pltpu.VMEM((PW * B * NH,), jnp.float32),
        pltpu.VMEM((EB, 16), jnp.float32),
        pltpu.VMEM((EB, 16), jnp.float32),
        pltpu.VMEM((EB, HF), jnp.bfloat16),
        pltpu.VMEM((EB, HF), jnp.bfloat16),
        pltpu.VMEM((B * HF,), jnp.float32),
        pltpu.VMEM((B * HF,), jnp.float32),
        pltpu.VMEM((HF,), jnp.float32),
        pltpu.SemaphoreType.DMA,
        pltpu.SemaphoreType.DMA,
        pltpu.SemaphoreType.DMA,
        pltpu.SemaphoreType.DMA,
        pltpu.SemaphoreType.DMA,
        pltpu.SemaphoreType.DMA,
    ],
)
def _sc_kern(colind_hbm, arf_hbm, ac2_hbm, h_hbm, biasp_hbm, out_hbm,
             ci_v, ar_v, acg0, acg1, hg0, hg1, outb0, outb1, bias_v,
             sa0, sh0, sa1, sh1, so0, so1):
    wid = lax.axis_index("s") * NC + lax.axis_index("c")
    cnt_extra = jnp.where(wid < EXTRA, 1, 0)
    start = BASE_CNT * wid + jnp.minimum(wid, EXTRA)
    copy_start = jnp.minimum(start, NB - PW)
    off = start - copy_start

    pltpu.sync_copy(colind_hbm.at[pl.ds(copy_start * EB, PW * EB)], ci_v)
    pltpu.sync_copy(arf_hbm.at[pl.ds(copy_start * B * NH, PW * B * NH)], ar_v)
    pltpu.sync_copy(biasp_hbm, bias_v)

    iota = lax.iota(jnp.int32, 16)
    pat8 = lax.bitwise_and(iota, 7)
    scat_base = pat8 * 16 + lax.shift_right_logical(iota, 3)
    bias_vs = [bias_v[pl.ds(k * 16, 16)] for k in range(NH)]

    def gdesc(l, acg_b, hg_b, sa, sh):
        idxs = ci_v.at[pl.ds((off + l) * EB, EB)]
        return (pltpu.make_async_copy(ac2_hbm.at[idxs], acg_b, sa),
                pltpu.make_async_copy(h_hbm.at[idxs], hg_b, sh))

    def compute_block(l, acg, hg, outb, so):
        @plsc.parallel_loop(0, B, unroll=2)
        def _(ln):
            r0 = ln * DEG
            arp = plsc.load_gather(ar_v, [pat8 + ((off + l) * B + ln) * NH])
            ex = []
            for e in range(DEG):
                v = arp + acg[r0 + e, :]
                ex.append(jnp.maximum(v, NEG_SLOPE * v))
            m = _tree_reduce(jnp.maximum, ex)
            ex = [jnp.exp(x - m) for x in ex]
            d = _tree_reduce(lambda a, b: a + b, ex)
            inv = 1.0 / (d + 1e-16)
            acc = [None] * NH
            for e in range(DEG):
                a = ex[e]
                for k in range(NH // 2):
                    packed = hg[r0 + e, pl.ds(k * 32, 32)]
                    va, vb = plsc.unpack(packed,
                                         format=plsc.PackFormat.INTERLEAVED)
                    if e == 0:
                        acc[2 * k] = a * va
                        acc[2 * k + 1] = a * vb
                    else:
                        acc[2 * k] = acc[2 * k] + a * va
                        acc[2 * k + 1] = acc[2 * k + 1] + a * vb
            for k in range(NH):
                plsc.store_scatter(outb, [scat_base + (2 * k + HF * ln)],
                                   bias_vs[k] + inv * acc[k])

        return pltpu.make_async_copy(
            outb, out_hbm.at[pl.ds((start + l) * (B * HF), B * HF)], so)

    def pair_body(i2, carry):
        l0 = 2 * i2
        d0 = gdesc(l0, acg0, hg0, sa0, sh0)
        d1 = gdesc(l0 + 1, acg1, hg1, sa1, sh1)
        for d in d0:
            d.start()
        for d in d1:
            d.start()
        for d in d0:
            d.wait()
        st0 = compute_block(l0, acg0, hg0, outb0, so0)
        st0.start()
        for d in d1:
            d.wait()
        st1 = compute_block(l0 + 1, acg1, hg1, outb1, so1)
        st1.start()
        st0.wait()
        st1.wait()
        return carry

    lax.fori_loop(0, BASE_CNT // 2, pair_body, 0)

    @pl.when(cnt_extra == 1)
    def _():
        d0 = gdesc(BASE_CNT, acg0, hg0, sa0, sh0)
        for d in d0:
            d.start()
        for d in d0:
            d.wait()
        st = compute_block(BASE_CNT, acg0, hg0, outb0, so0)
        st.start()
        st.wait()


def kernel(row_ptr, col_ind, col_ptr, row_ind, permute, feat, W,
           attn_l, attn_r, bias):
    j = jnp.arange(HF, dtype=jnp.int32)
    permc = (j & 7) * 16 + (j >> 3)
    kg = j >> 5
    t = j & 31
    jj = t >> 1
    odd = t & 1
    permc2 = (jj & 7) * 16 + (4 * kg + 2 * odd + (jj >> 3))
    Wp = W[:, permc2]
    al = attn_l.reshape(NH, NF)
    ar_ = attn_r.reshape(NH, NF)
    hh2 = permc2 >> 4
    ff2 = permc2 & 15
    Alp = jnp.zeros((HF, NH), jnp.float32).at[j, hh2].set(al[hh2, ff2])
    Arp = jnp.zeros((HF, NH), jnp.float32).at[j, hh2].set(ar_[hh2, ff2])
    bias_p = bias[permc]

    h_perm, ar, ac2 = _tc_call(feat, Wp, Alp, Arp)
    out_flat = _sc_kern(col_ind, ar.reshape(-1), ac2, h_perm, bias_p)
    return out_flat.reshape(N, NH, NF)


# speedup vs baseline: 1.1329x; 1.1329x over previous
import functools

import jax
import jax.numpy as jnp
from jax import lax
from jax.experimental import pallas as pl
from jax.experimental.pallas import tpu as pltpu
from jax.experimental.pallas import tpu_sc as plsc

N = 10000
DEG = 32
E = N * DEG
HF = 128
NH = 8
NF = 16
NEG_SLOPE = 0.2

ROWS_TC = 2000

NC, NS = 2, 16
NW = NC * NS
B = 4
EB = B * DEG
NB = N // B
BASE_CNT = NB // NW
EXTRA = NB % NW
PW = BASE_CNT + 1


def _tc_body(feat_ref, wp_ref, alp_ref, arp_ref, h_ref, ar_ref, ac2_ref):
    h = jnp.dot(feat_ref[...], wp_ref[...],
                preferred_element_type=jnp.float32,
                precision=lax.Precision.HIGHEST)
    h_ref[...] = h.astype(jnp.bfloat16)
    ar_ref[...] = jnp.dot(h, alp_ref[...],
                          preferred_element_type=jnp.float32,
                          precision=lax.Precision.HIGHEST)
    acv = jnp.dot(h, arp_ref[...],
                  preferred_element_type=jnp.float32,
                  precision=lax.Precision.HIGHEST)
    ac2_ref[...] = jnp.concatenate([acv, acv], axis=1)


_tc_call = pl.pallas_call(
    _tc_body,
    grid=(N // ROWS_TC,),
    in_specs=[
        pl.BlockSpec((ROWS_TC, HF), lambda i: (i, 0)),
        pl.BlockSpec((HF, HF), lambda i: (0, 0)),
        pl.BlockSpec((HF, NH), lambda i: (0, 0)),
        pl.BlockSpec((HF, NH), lambda i: (0, 0)),
    ],
    out_specs=[
        pl.BlockSpec((ROWS_TC, HF), lambda i: (i, 0)),
        pl.BlockSpec((ROWS_TC, NH), lambda i: (i, 0)),
        pl.BlockSpec((ROWS_TC, 2 * NH), lambda i: (i, 0)),
    ],
    out_shape=[
        jax.ShapeDtypeStruct((N, HF), jnp.bfloat16),
        jax.ShapeDtypeStruct((N, NH), jnp.float32),
        jax.ShapeDtypeStruct((N, 2 * NH), jnp.float32),
    ],
)


def _tree_reduce(op, xs):
    xs = list(xs)
    while len(xs) > 1:
        nxt = [op(xs[i], xs[i + 1]) for i in range(0, len(xs) - 1, 2)]
        if len(xs) % 2:
            nxt.append(xs[-1])
        xs = nxt
    return xs[0]


@functools.partial(
    pl.kernel,
    mesh=plsc.VectorSubcoreMesh(core_axis_name="c", subcore_axis_name="s"),
    out_type=jax.ShapeDtypeStruct((N * HF,), jnp.float32),
    compiler_params=pltpu.CompilerParams(needs_layout_passes=False,
                                         use_tc_tiling_on_sc=False),
    scratch_types=[
        pltpu.VMEM((PW * EB,), jnp.int32),
        pltpu.VMEM((PW * B * NH,), jnp.float32),
        pltpu.VMEM((EB, 16), jnp.float32),
        pltpu.VMEM((EB, 16), jnp.float32),
        pltpu.VMEM((EB, HF), jnp.bfloat16),
        pltpu.VMEM((EB, HF), jnp.bfloat16),
        pltpu.VMEM((B * HF,), jnp.float32),
        pltpu.VMEM((HF,), jnp.float32),
        pltpu.SemaphoreType.DMA,
        pltpu.SemaphoreType.DMA,
        pltpu.SemaphoreType.DMA,
        pltpu.SemaphoreType.DMA,
    ],
)
def _sc_kern(colind_hbm, arf_hbm, ac2_hbm, h_hbm, biasp_hbm, out_hbm,
             ci_v, ar_v, acg0, acg1, hg0, hg1, outb, bias_v,
             sa0, sh0, sa1, sh1):
    wid = lax.axis_index("s") * NC + lax.axis_index("c")
    cnt_extra = jnp.where(wid < EXTRA, 1, 0)
    start = BASE_CNT * wid + jnp.minimum(wid, EXTRA)
    copy_start = jnp.minimum(start, NB - PW)
    off = start - copy_start

    pltpu.sync_copy(colind_hbm.at[pl.ds(copy_start * EB, PW * EB)], ci_v)
    pltpu.sync_copy(arf_hbm.at[pl.ds(copy_start * B * NH, PW * B * NH)], ar_v)
    pltpu.sync_copy(biasp_hbm, bias_v)

    iota = lax.iota(jnp.int32, 16)
    pat8 = lax.bitwise_and(iota, 7)
    scat_base = pat8 * 16 + lax.shift_right_logical(iota, 3)
    bias_vs = [bias_v[pl.ds(k * 16, 16)] for k in range(NH)]

    def gdesc(l, acg_b, hg_b, sa, sh):
        idxs = ci_v.at[pl.ds((off + l) * EB, EB)]
        return (pltpu.make_async_copy(ac2_hbm.at[idxs], acg_b, sa),
                pltpu.make_async_copy(h_hbm.at[idxs], hg_b, sh))

    def compute_block(l, acg, hg):
        @plsc.parallel_loop(0, B, unroll=2)
        def _(ln):
            r0 = ln * DEG
            arp = plsc.load_gather(ar_v, [pat8 + ((off + l) * B + ln) * NH])
            ex = []
            for e in range(DEG):
                v = arp + acg[r0 + e, :]
                ex.append(jnp.maximum(v, NEG_SLOPE * v))
            m = _tree_reduce(jnp.maximum, ex)
            ex = [jnp.exp(x - m) for x in ex]
            d = _tree_reduce(lambda a, b: a + b, ex)
            inv = 1.0 / (d + 1e-16)
            acc = [None] * NH
            for e in range(DEG):
                a = ex[e]
                for k in range(NH // 2):
                    packed = hg[r0 + e, pl.ds(k * 32, 32)]
                    va, vb = plsc.unpack(packed,
                                         format=plsc.PackFormat.INTERLEAVED)
                    if e == 0:
                        acc[2 * k] = a * va
                        acc[2 * k + 1] = a * vb
                    else:
                        acc[2 * k] = acc[2 * k] + a * va
                        acc[2 * k + 1] = acc[2 * k + 1] + a * vb
            for k in range(NH):
                plsc.store_scatter(outb, [scat_base + (2 * k + HF * ln)],
                                   bias_vs[k] + inv * acc[k])

        pltpu.sync_copy(outb,
                        out_hbm.at[pl.ds((start + l) * (B * HF), B * HF)])

    def pair_body(i2, carry):
        l0 = 2 * i2
        d0 = gdesc(l0, acg0, hg0, sa0, sh0)
        d1 = gdesc(l0 + 1, acg1, hg1, sa1, sh1)
        for d in d0:
            d.start()
        for d in d1:
            d.start()
        for d in d0:
            d.wait()
        compute_block(l0, acg0, hg0)
        for d in d1:
            d.wait()
        compute_block(l0 + 1, acg1, hg1)
        return carry

    lax.fori_loop(0, BASE_CNT // 2, pair_body, 0)

    @pl.when(cnt_extra == 1)
    def _():
        d0 = gdesc(BASE_CNT, acg0, hg0, sa0, sh0)
        for d in d0:
            d.start()
        for d in d0:
            d.wait()
        compute_block(BASE_CNT, acg0, hg0)


def kernel(row_ptr, col_ind, col_ptr, row_ind, permute, feat, W,
           attn_l, attn_r, bias):
    j = jnp.arange(HF, dtype=jnp.int32)
    permc = (j & 7) * 16 + (j >> 3)
    kg = j >> 5
    t = j & 31
    jj = t >> 1
    odd = t & 1
    permc2 = (jj & 7) * 16 + (4 * kg + 2 * odd + (jj >> 3))
    Wp = W[:, permc2]
    al = attn_l.reshape(NH, NF)
    ar_ = attn_r.reshape(NH, NF)
    hh2 = permc2 >> 4
    ff2 = permc2 & 15
    Alp = jnp.zeros((HF, NH), jnp.float32).at[j, hh2].set(al[hh2, ff2])
    Arp = jnp.zeros((HF, NH), jnp.float32).at[j, hh2].set(ar_[hh2, ff2])
    bias_p = bias[permc]

    h_perm, ar, ac2 = _tc_call(feat, Wp, Alp, Arp)
    out_flat = _sc_kern(col_ind, ar.reshape(-1), ac2, h_perm, bias_p)
    return out_flat.reshape(N, NH, NF)
